# baseline (speedup 1.0000x reference)
import jax
import jax.numpy as jnp
from jax.experimental import pallas as pl
from jax.experimental.pallas import tpu as pltpu

_S, _N, _F, _H, _R = 8, 1024, 128, 256, 4


def _leaky(v):
    return jnp.where(v >= 0, v, 0.01 * v)


def _dot(a, b):
    return jnp.dot(a, b, preferred_element_type=jnp.float32)


def _rgcn_last_step_kernel(
    x_ref, adj_ref,
    fc1_w_ref, fc1_b_ref, fc2_w_ref, fc2_b_ref,
    fc3_w_ref, fc3_b_ref, fc4_w_ref, fc4_b_ref,
    g0_ws_ref, g0_wr_ref, g0_b_ref,
    g1_ws_ref, g1_wr_ref, g1_b_ref,
    out_ref,
    h_ref, msg_ref,
):
    i = pl.program_id(0)
    r = jax.lax.rem(i, _R)

    @pl.when(i == 0)
    def _fc_head():
        x = x_ref[0, 0]
        h = _leaky(_dot(x, fc1_w_ref[...]) + fc1_b_ref[...])
        h = _leaky(_dot(h, fc2_w_ref[...]) + fc2_b_ref[...])
        h_ref[...] = h
        msg_ref[...] = _dot(h, g0_ws_ref[...]) + g0_b_ref[...]

    h = h_ref[...]
    adj = adj_ref[0, 0, 0]
    deg = jnp.sum(adj, axis=1, keepdims=True) + 1e-6
    agg = _dot(adj, h) / deg

    @pl.when(i < _R)
    def _acc0():
        msg_ref[...] = msg_ref[...] + _dot(agg, g0_wr_ref[r])

    @pl.when(i >= _R)
    def _acc1():
        msg_ref[...] = msg_ref[...] + _dot(agg, g1_wr_ref[r])

    @pl.when(i == _R - 1)
    def _finish_layer0():
        h1 = _leaky(msg_ref[...])
        h_ref[...] = h1
        msg_ref[...] = _dot(h1, g1_ws_ref[...]) + g1_b_ref[...]

    @pl.when(i == 2 * _R - 1)
    def _finish():
        h2 = _leaky(msg_ref[...])
        o = _leaky(_dot(h2, fc3_w_ref[...]) + fc3_b_ref[...])
        y = jnp.sum(o * fc4_w_ref[...], axis=1, keepdims=True) + fc4_b_ref[0, 0]
        out_ref[0] = y


def kernel(x, adjs, edgenum, fc1_w, fc1_b, fc2_w, fc2_b, fc3_w, fc3_b,
           fc4_w, fc4_b, g0_wself, g0_wrel, g0_b, g1_wself, g1_wrel, g1_b):
    del edgenum
    last = _S - 1

    def full(shape):
        return pl.BlockSpec(shape, lambda i: tuple(0 for _ in shape))

    in_specs = [
        pl.BlockSpec((1, 1, _N, _F), lambda i: (0, last, 0, 0)),
        pl.BlockSpec((1, 1, 1, _N, _N),
                     lambda i: (0, last, jax.lax.rem(i, _R), 0, 0)),
        full((_F, _H)), full((1, _H)),
        full((_H, _H)), full((1, _H)),
        full((_H, _H)), full((1, _H)),
        full((1, _H)), full((1, 1)),
        full((_H, _H)), full((_R, _H, _H)), full((1, _H)),
        full((_H, _H)), full((_R, _H, _H)), full((1, _H)),
    ]

    out = pl.pallas_call(
        _rgcn_last_step_kernel,
        out_shape=jax.ShapeDtypeStruct((1, _N, 1), jnp.float32),
        grid=(2 * _R,),
        in_specs=in_specs,
        out_specs=pl.BlockSpec((1, _N, 1), lambda i: (0, 0, 0)),
        scratch_shapes=[
            pltpu.VMEM((_N, _H), jnp.float32),
            pltpu.VMEM((_N, _H), jnp.float32),
        ],
        compiler_params=pltpu.CompilerParams(
            vmem_limit_bytes=100 * 1024 * 1024,
        ),
    )(
        x, adjs,
        fc1_w, fc1_b.reshape(1, _H), fc2_w, fc2_b.reshape(1, _H),
        fc3_w, fc3_b.reshape(1, _H),
        fc4_w.reshape(1, _H), fc4_b.reshape(1, 1),
        g0_wself, g0_wrel, g0_b.reshape(1, _H),
        g1_wself, g1_wrel, g1_b.reshape(1, _H),
    )
    return out


# speedup vs baseline: 1.0956x; 1.0956x over previous
import jax
import jax.numpy as jnp
from jax.experimental import pallas as pl
from jax.experimental.pallas import tpu as pltpu

_S, _N, _F, _H, _R = 8, 1024, 128, 256, 4


def _leaky(v):
    return jnp.where(v >= 0, v, 0.01 * v)


def _dot(a, b):
    return jnp.dot(a, b, preferred_element_type=jnp.float32)


def _rgcn_last_step_kernel(
    x_ref, adj_ref,
    fc1_w_ref, fc1_b_ref, fc2_w_ref, fc2_b_ref,
    fc3_w_ref, fc3_b_ref, fc4_w_ref, fc4_b_ref,
    g0_ws_ref, g0_wr_ref, g0_b_ref,
    g1_ws_ref, g1_wr_ref, g1_b_ref,
    out_ref,
):
    x = x_ref[0, 0]
    h = _leaky(_dot(x, fc1_w_ref[...]) + fc1_b_ref[...])
    h = _leaky(_dot(h, fc2_w_ref[...]) + fc2_b_ref[...])

    inv_deg = [
        1.0 / (jnp.sum(adj_ref[0, 0, r], axis=1, keepdims=True) + 1e-6)
        for r in range(_R)
    ]

    def rgcn(h, ws_ref, wr_ref, b_ref):
        acc = _dot(h, ws_ref[...]) + b_ref[...]
        for r in range(_R):
            agg = _dot(adj_ref[0, 0, r], h) * inv_deg[r]
            acc = acc + _dot(agg, wr_ref[r])
        return _leaky(acc)

    h = rgcn(h, g0_ws_ref, g0_wr_ref, g0_b_ref)
    h = rgcn(h, g1_ws_ref, g1_wr_ref, g1_b_ref)

    o = _leaky(_dot(h, fc3_w_ref[...]) + fc3_b_ref[...])
    y = jnp.sum(o * fc4_w_ref[...], axis=1, keepdims=True) + fc4_b_ref[0, 0]
    out_ref[0] = y


def kernel(x, adjs, edgenum, fc1_w, fc1_b, fc2_w, fc2_b, fc3_w, fc3_b,
           fc4_w, fc4_b, g0_wself, g0_wrel, g0_b, g1_wself, g1_wrel, g1_b):
    del edgenum
    last = _S - 1

    def full(shape):
        return pl.BlockSpec(shape, lambda i: tuple(0 for _ in shape))

    in_specs = [
        pl.BlockSpec((1, 1, _N, _F), lambda i: (0, last, 0, 0)),
        pl.BlockSpec((1, 1, _R, _N, _N), lambda i: (0, last, 0, 0, 0)),
        full((_F, _H)), full((1, _H)),
        full((_H, _H)), full((1, _H)),
        full((_H, _H)), full((1, _H)),
        full((1, _H)), full((1, 1)),
        full((_H, _H)), full((_R, _H, _H)), full((1, _H)),
        full((_H, _H)), full((_R, _H, _H)), full((1, _H)),
    ]

    out = pl.pallas_call(
        _rgcn_last_step_kernel,
        out_shape=jax.ShapeDtypeStruct((1, _N, 1), jnp.float32),
        grid=(1,),
        in_specs=in_specs,
        out_specs=pl.BlockSpec((1, _N, 1), lambda i: (0, 0, 0)),
        compiler_params=pltpu.CompilerParams(
            vmem_limit_bytes=100 * 1024 * 1024,
        ),
    )(
        x, adjs,
        fc1_w, fc1_b.reshape(1, _H), fc2_w, fc2_b.reshape(1, _H),
        fc3_w, fc3_b.reshape(1, _H),
        fc4_w.reshape(1, _H), fc4_b.reshape(1, 1),
        g0_wself, g0_wrel, g0_b.reshape(1, _H),
        g1_wself, g1_wrel, g1_b.reshape(1, _H),
    )
    return out


# speedup vs baseline: 1.0967x; 1.0010x over previous
import jax
import jax.numpy as jnp
from jax.experimental import pallas as pl
from jax.experimental.pallas import tpu as pltpu

_S, _N, _F, _H, _R = 8, 1024, 128, 256, 4


def _leaky(v):
    return jnp.where(v >= 0, v, 0.01 * v)


def _dot(a, b):
    return jnp.dot(a, b, preferred_element_type=jnp.float32)


def _rgcn_last_step_kernel(
    x_ref, adj0_ref, adj1_ref, adj2_ref, adj3_ref,
    fc1_w_ref, fc1_b_ref, fc2_w_ref, fc2_b_ref,
    fc3_w_ref, fc3_b_ref, fc4_w_ref, fc4_b_ref,
    g0_ws_ref, g0_wr_ref, g0_b_ref,
    g1_ws_ref, g1_wr_ref, g1_b_ref,
    out_ref,
):
    adj_refs = (adj0_ref, adj1_ref, adj2_ref, adj3_ref)
    x = x_ref[0, 0]
    h = _leaky(_dot(x, fc1_w_ref[...]) + fc1_b_ref[...])
    h = _leaky(_dot(h, fc2_w_ref[...]) + fc2_b_ref[...])

    inv_deg = [
        1.0 / (jnp.sum(adj_refs[r][0, 0, 0], axis=1, keepdims=True) + 1e-6)
        for r in range(_R)
    ]

    def rgcn(h, ws_ref, wr_ref, b_ref):
        acc = _dot(h, ws_ref[...]) + b_ref[...]
        for r in range(_R):
            agg = _dot(adj_refs[r][0, 0, 0], h) * inv_deg[r]
            acc = acc + _dot(agg, wr_ref[r])
        return _leaky(acc)

    h = rgcn(h, g0_ws_ref, g0_wr_ref, g0_b_ref)
    h = rgcn(h, g1_ws_ref, g1_wr_ref, g1_b_ref)

    o = _leaky(_dot(h, fc3_w_ref[...]) + fc3_b_ref[...])
    y = jnp.sum(o * fc4_w_ref[...], axis=1, keepdims=True) + fc4_b_ref[0, 0]
    out_ref[0] = y


def kernel(x, adjs, edgenum, fc1_w, fc1_b, fc2_w, fc2_b, fc3_w, fc3_b,
           fc4_w, fc4_b, g0_wself, g0_wrel, g0_b, g1_wself, g1_wrel, g1_b):
    del edgenum
    last = _S - 1

    def full(shape):
        return pl.BlockSpec(shape, lambda i: tuple(0 for _ in shape))

    in_specs = [
        pl.BlockSpec((1, 1, _N, _F), lambda i: (0, last, 0, 0)),
        pl.BlockSpec((1, 1, 1, _N, _N), lambda i: (0, last, 0, 0, 0)),
        pl.BlockSpec((1, 1, 1, _N, _N), lambda i: (0, last, 1, 0, 0)),
        pl.BlockSpec((1, 1, 1, _N, _N), lambda i: (0, last, 2, 0, 0)),
        pl.BlockSpec((1, 1, 1, _N, _N), lambda i: (0, last, 3, 0, 0)),
        full((_F, _H)), full((1, _H)),
        full((_H, _H)), full((1, _H)),
        full((_H, _H)), full((1, _H)),
        full((1, _H)), full((1, 1)),
        full((_H, _H)), full((_R, _H, _H)), full((1, _H)),
        full((_H, _H)), full((_R, _H, _H)), full((1, _H)),
    ]

    out = pl.pallas_call(
        _rgcn_last_step_kernel,
        out_shape=jax.ShapeDtypeStruct((1, _N, 1), jnp.float32),
        grid=(1,),
        in_specs=in_specs,
        out_specs=pl.BlockSpec((1, _N, 1), lambda i: (0, 0, 0)),
        compiler_params=pltpu.CompilerParams(
            vmem_limit_bytes=100 * 1024 * 1024,
        ),
    )(
        x, adjs, adjs, adjs, adjs,
        fc1_w, fc1_b.reshape(1, _H), fc2_w, fc2_b.reshape(1, _H),
        fc3_w, fc3_b.reshape(1, _H),
        fc4_w.reshape(1, _H), fc4_b.reshape(1, 1),
        g0_wself, g0_wrel, g0_b.reshape(1, _H),
        g1_wself, g1_wrel, g1_b.reshape(1, _H),
    )
    return out
